# baseline (speedup 1.0000x reference)
import functools

import jax
import jax.numpy as jnp
from jax import lax
from jax.experimental import pallas as pl
from jax.experimental.pallas import tpu as pltpu
from jax.experimental.pallas import tpu_sc as plsc

N = 10000
E = 160000
D = 256
NC = 2
NS = 16
NW = NC * NS
RPT = 320
TRASH = RPT
CAP = 8192
Q = 64
CAP_CLAMP = CAP - 2 * Q - 16
CH = 8000
NCHUNK = E // CH
GPC = CH // 16
MMB = 1000
BF = jnp.bfloat16

@functools.cache
def _mesh():
    return plsc.VectorSubcoreMesh(
        core_axis_name="c", subcore_axis_name="s",
        num_cores=NC, num_subcores=NS)


def _mm_body(x_ref, w_ref, b_ref, a_ref, c_ref):
    s = jnp.dot(x_ref[...], w_ref[...], preferred_element_type=jnp.float32)
    a_ref[...] = s[:, :D].astype(BF)
    c_ref[...] = s[:, D:] + b_ref[...]


def _matmul_ab(xin, wcat_t, b2d):
    return pl.pallas_call(
        _mm_body,
        grid=(N // MMB,),
        in_specs=[
            pl.BlockSpec((MMB, D), lambda i: (i, 0)),
            pl.BlockSpec((D, 2 * D), lambda i: (0, 0)),
            pl.BlockSpec((1, D), lambda i: (0, 0)),
        ],
        out_specs=[
            pl.BlockSpec((MMB, D), lambda i: (i, 0)),
            pl.BlockSpec((MMB, D), lambda i: (i, 0)),
        ],
        out_shape=[jax.ShapeDtypeStruct((N, D), BF),
                   jax.ShapeDtypeStruct((N, D), jnp.float32)],
    )(xin, wcat_t, b2d)


def _bucket_body(src_hbm, dst_hbm, msrc_hbm, mdst_hbm, cnt_hbm,
                 sbuf0, dbuf0, sbuf1, dbuf1, msrc_b, mdst_b, cbuf,
                 csem0, csem1):
    wid = lax.axis_index("s") * NC + lax.axis_index("c")
    lo = wid * RPT
    lanes = lax.iota(jnp.int32, 16)

    def cfire(ci, sb, db, sem):
        ci = jnp.minimum(ci, NCHUNK - 1)
        pltpu.make_async_copy(src_hbm.at[pl.ds(ci * CH, CH)], sb, sem).start()
        pltpu.make_async_copy(dst_hbm.at[pl.ds(ci * CH, CH)], db, sem).start()

    def cwait(sb, db, sem):
        pltpu.make_async_copy(src_hbm.at[pl.ds(0, CH)], sb, sem).wait()
        pltpu.make_async_copy(src_hbm.at[pl.ds(0, CH)], db, sem).wait()

    def scan_chunk(sb, db, off):
        def group_body(g, off):
            vs = sb[pl.ds(g * 16, 16)]
            vd = db[pl.ds(g * 16, 16)]
            m = (vd >= lo) & (vd < lo + RPT)
            mi = m.astype(jnp.int32)
            incl = plsc.cumsum(mi)
            idx = off + incl - mi
            plsc.store_scatter(msrc_b, [idx], vs, mask=m)
            plsc.store_scatter(mdst_b, [idx], vd - lo, mask=m)
            pc = plsc.all_reduce_population_count(m)
            return jnp.minimum(off + pc[0], CAP_CLAMP)

        return lax.fori_loop(0, GPC, group_body, off)

    cfire(jnp.int32(0), sbuf0, dbuf0, csem0)

    def chunk_pair(pc, off):
        cwait(sbuf0, dbuf0, csem0)
        cfire(2 * pc + 1, sbuf1, dbuf1, csem1)
        off = scan_chunk(sbuf0, dbuf0, off)
        cwait(sbuf1, dbuf1, csem1)
        cfire(2 * pc + 2, sbuf0, dbuf0, csem0)
        off = scan_chunk(sbuf1, dbuf1, off)
        return off

    off = lax.fori_loop(0, NCHUNK // 2, chunk_pair, jnp.int32(0))
    cwait(sbuf0, dbuf0, csem0)
    zeros = jnp.zeros((16,), jnp.int32)
    trash = jnp.full((16,), TRASH, jnp.int32)
    for t in range(2 * Q // 16):
        plsc.store_scatter(msrc_b, [off + 16 * t + lanes], zeros)
        plsc.store_scatter(mdst_b, [off + 16 * t + lanes], trash)
    cbuf[...] = jnp.full((16,), off, jnp.int32)
    pltpu.sync_copy(msrc_b, msrc_hbm.at[pl.ds(wid * CAP, CAP)])
    pltpu.sync_copy(mdst_b, mdst_hbm.at[pl.ds(wid * CAP, CAP)])
    pltpu.sync_copy(cbuf, cnt_hbm.at[pl.ds(wid * 16, 16)])


@functools.cache
def _bucket():
    return pl.kernel(
        _bucket_body,
        out_type=[
            jax.ShapeDtypeStruct((NW * CAP,), jnp.int32),
            jax.ShapeDtypeStruct((NW * CAP,), jnp.int32),
            jax.ShapeDtypeStruct((NW * 16,), jnp.int32),
        ],
        mesh=_mesh(),
        compiler_params=pltpu.CompilerParams(needs_layout_passes=False),
        scratch_types=[
            pltpu.VMEM((CH,), jnp.int32),
            pltpu.VMEM((CH,), jnp.int32),
            pltpu.VMEM((CH,), jnp.int32),
            pltpu.VMEM((CH,), jnp.int32),
            pltpu.VMEM((CAP,), jnp.int32),
            pltpu.VMEM((CAP,), jnp.int32),
            pltpu.VMEM((16,), jnp.int32),
            pltpu.SemaphoreType.DMA,
            pltpu.SemaphoreType.DMA,
        ],
    )


def _segmax_body(a_hbm, b_hbm, msrc_hbm, mdst_hbm, cnt_hbm, out_hbm,
                 msrc_v, mdst_v, cnt_v, acc, acc1, rb0, rb1, bbuf, sem0, sem1):
    wid = lax.axis_index("s") * NC + lax.axis_index("c")
    lo = wid * RPT
    rows_here = jnp.minimum(RPT, N - lo)

    pltpu.sync_copy(msrc_hbm.at[pl.ds(wid * CAP, CAP)], msrc_v)
    pltpu.sync_copy(mdst_hbm.at[pl.ds(wid * CAP, CAP)], mdst_v)
    pltpu.sync_copy(cnt_hbm.at[pl.ds(wid * 16, 16)], cnt_v)
    count = jnp.minimum(jnp.max(cnt_v[...]), CAP_CLAMP)
    nq = (count + Q - 1) // Q
    npairs = (nq + 1) // 2
    qmax = jnp.maximum(nq - 1, 0)

    ninf = plsc.bitcast(jnp.full((32,), -jnp.inf, BF), jnp.int32)

    def init_body(i, _):
        acc[pl.ds(i * 16, 16)] = ninf
        acc1[pl.ds(i * 16, 16)] = ninf
        return 0

    lax.fori_loop(0, (RPT + 1) * D // 32, init_body, 0)

    def fire(q, rbuf, sem):
        idx = msrc_v.at[pl.ds(q * Q, Q)]
        pltpu.make_async_copy(a_hbm.at[idx], rbuf, sem).start()

    def wait(rbuf, sem):
        pltpu.make_async_copy(a_hbm.at[pl.ds(0, Q)], rbuf, sem).wait()

    def process(q, rbuf):
        for kk in range(Q // 16):
            vd = mdst_v[pl.ds(q * Q + kk * 16, 16)]
            for k in range(16):
                base = vd[k] * (D // 2)
                ac = acc if k % 2 == 0 else acc1
                for j in range(8):
                    sl = pl.ds(base + j * 16, 16)
                    av = plsc.bitcast(ac[sl], BF)
                    rv = plsc.bitcast(
                        rbuf[kk * 16 + k, pl.ds(j * 16, 16)], BF)
                    ac[sl] = plsc.bitcast(jnp.maximum(av, rv), jnp.int32)

    fire(jnp.int32(0), rb0, sem0)

    def pair_body(p, _):
        q1 = jnp.minimum(2 * p + 1, qmax)
        q2 = jnp.minimum(2 * p + 2, qmax)
        wait(rb0, sem0)
        fire(q1, rb1, sem1)
        process(2 * p, rb0)
        wait(rb1, sem1)
        fire(q2, rb0, sem0)
        process(q1, rb1)
        return 0

    lax.fori_loop(0, npairs, pair_body, 0)
    wait(rb0, sem0)

    def out_chunk(t, _):
        base_r = jnp.minimum(lo + t * 32, lo + rows_here - 32)
        pltpu.sync_copy(b_hbm.at[pl.ds(base_r, 32)], bbuf)
        arow0 = (base_r - lo) * D // 2

        def row_body(r, _):
            for j in range(8):
                wsl = pl.ds(arow0 + (r * D + j * 32) // 2, 16)
                mx = jnp.maximum(plsc.bitcast(acc[wsl], BF),
                                 plsc.bitcast(acc1[wsl], BF))
                ev, od = plsc.unpack(
                    mx, format=plsc.PackFormat.INTERLEAVED,
                    preferred_element_type=jnp.float32)
                bse = (r, pl.ds(j * 32, 16))
                bso = (r, pl.ds(j * 32 + 16, 16))
                bbuf[bse] = jnp.maximum(bbuf[bse] + ev, 0.0)
                bbuf[bso] = jnp.maximum(bbuf[bso] + od, 0.0)
            return 0

        lax.fori_loop(0, 32, row_body, 0)
        pltpu.sync_copy(bbuf, out_hbm.at[pl.ds(base_r, 32)])
        return 0

    lax.fori_loop(0, RPT // 32, out_chunk, 0)


@functools.cache
def _segmax():
    return pl.kernel(
        _segmax_body,
        out_type=jax.ShapeDtypeStruct((N, D), jnp.float32),
        mesh=_mesh(),
        compiler_params=pltpu.CompilerParams(needs_layout_passes=False),
        scratch_types=[
            pltpu.VMEM((CAP,), jnp.int32),
            pltpu.VMEM((CAP,), jnp.int32),
            pltpu.VMEM((16,), jnp.int32),
            pltpu.VMEM(((RPT + 1) * D // 2,), jnp.int32),
            pltpu.VMEM(((RPT + 1) * D // 2,), jnp.int32),
            pltpu.VMEM((Q, D // 2), jnp.int32),
            pltpu.VMEM((Q, D // 2), jnp.int32),
            pltpu.VMEM((32, D), jnp.float32),
            pltpu.SemaphoreType.DMA,
            pltpu.SemaphoreType.DMA,
        ],
    )


def _to_i32(a):
    return lax.bitcast_convert_type(a.reshape(N, D // 2, 2), jnp.int32)


import numpy as _np
_PI = _np.arange(D).reshape(8, 16, 2).transpose(0, 2, 1).reshape(D)


def _unperm(h):
    return h.reshape(N, 8, 2, 16).transpose(0, 1, 3, 2).reshape(N, D)


def _prep_w(W):
    wa = W[:, :D]
    wc = W[:, D:] - wa
    wcat_t = jnp.concatenate([wa, wc], axis=0).T
    return jnp.concatenate([wcat_t[:, :D], wcat_t[:, D:][:, _PI]], axis=1)


def kernel(x, edge_index, W1, b1, W2, b2):
    src = edge_index[0]
    dst = edge_index[1]
    msrc, mdst, counts = _bucket()(src, dst)
    a1, bp1 = _matmul_ab(x, _prep_w(W1), b1[_PI].reshape(1, D))
    h1 = _unperm(_segmax()(_to_i32(a1), bp1, msrc, mdst, counts))
    a2, bp2 = _matmul_ab(h1, _prep_w(W2), b2[_PI].reshape(1, D))
    h2 = _unperm(_segmax()(_to_i32(a2), bp2, msrc, mdst, counts))
    return jnp.concatenate([x, h1, h2], axis=-1)


# speedup vs baseline: 1.1360x; 1.1360x over previous
import functools

import jax
import jax.numpy as jnp
from jax import lax
from jax.experimental import pallas as pl
from jax.experimental.pallas import tpu as pltpu
from jax.experimental.pallas import tpu_sc as plsc

N = 10000
E = 160000
D = 256
NC = 2
NS = 16
NW = NC * NS
RPT = 320
TRASH = RPT
CAP = 8192
Q = 64
CAP_CLAMP = CAP - 2 * Q - 16
CH = 8000
NCHUNK = E // CH
GPC = CH // 16
MMB = 1000
BF = jnp.bfloat16

@functools.cache
def _mesh():
    return plsc.VectorSubcoreMesh(
        core_axis_name="c", subcore_axis_name="s",
        num_cores=NC, num_subcores=NS)


def _mm_body(x_ref, w_ref, b_ref, a_ref, c_ref):
    s = jnp.dot(x_ref[...], w_ref[...], preferred_element_type=jnp.float32)
    a_ref[...] = s[:, :D].astype(BF)
    c_ref[...] = s[:, D:] + b_ref[...]


def _matmul_ab(xin, wcat_t, b2d):
    return pl.pallas_call(
        _mm_body,
        grid=(N // MMB,),
        in_specs=[
            pl.BlockSpec((MMB, D), lambda i: (i, 0)),
            pl.BlockSpec((D, 2 * D), lambda i: (0, 0)),
            pl.BlockSpec((1, D), lambda i: (0, 0)),
        ],
        out_specs=[
            pl.BlockSpec((MMB, D), lambda i: (i, 0)),
            pl.BlockSpec((MMB, D), lambda i: (i, 0)),
        ],
        out_shape=[jax.ShapeDtypeStruct((N, D), BF),
                   jax.ShapeDtypeStruct((N, D), jnp.float32)],
    )(xin, wcat_t, b2d)


def _bucket_body(src_hbm, dst_hbm, msrc_hbm, mdst_hbm, cnt_hbm,
                 sbuf0, dbuf0, sbuf1, dbuf1, msrc_b, mdst_b, cbuf,
                 csem0, csem1):
    wid = lax.axis_index("s") * NC + lax.axis_index("c")
    lo = wid * RPT
    lanes = lax.iota(jnp.int32, 16)

    def cfire(ci, sb, db, sem):
        ci = jnp.minimum(ci, NCHUNK - 1)
        pltpu.make_async_copy(src_hbm.at[pl.ds(ci * CH, CH)], sb, sem).start()
        pltpu.make_async_copy(dst_hbm.at[pl.ds(ci * CH, CH)], db, sem).start()

    def cwait(sb, db, sem):
        pltpu.make_async_copy(src_hbm.at[pl.ds(0, CH)], sb, sem).wait()
        pltpu.make_async_copy(src_hbm.at[pl.ds(0, CH)], db, sem).wait()

    def scan_chunk(sb, db, off):
        def group_body(g, off):
            vs = sb[pl.ds(g * 16, 16)]
            vd = db[pl.ds(g * 16, 16)]
            m = (vd >= lo) & (vd < lo + RPT)
            mi = m.astype(jnp.int32)
            incl = plsc.cumsum(mi)
            idx = off + incl - mi
            plsc.store_scatter(msrc_b, [idx], vs, mask=m)
            plsc.store_scatter(mdst_b, [idx], vd - lo, mask=m)
            pc = plsc.all_reduce_population_count(m)
            return jnp.minimum(off + pc[0], CAP_CLAMP)

        return lax.fori_loop(0, GPC, group_body, off)

    cfire(jnp.int32(0), sbuf0, dbuf0, csem0)

    def chunk_pair(pc, off):
        cwait(sbuf0, dbuf0, csem0)
        cfire(2 * pc + 1, sbuf1, dbuf1, csem1)
        off = scan_chunk(sbuf0, dbuf0, off)
        cwait(sbuf1, dbuf1, csem1)
        cfire(2 * pc + 2, sbuf0, dbuf0, csem0)
        off = scan_chunk(sbuf1, dbuf1, off)
        return off

    off = lax.fori_loop(0, NCHUNK // 2, chunk_pair, jnp.int32(0))
    cwait(sbuf0, dbuf0, csem0)
    zeros = jnp.zeros((16,), jnp.int32)
    trash = jnp.full((16,), TRASH, jnp.int32)
    for t in range(2 * Q // 16):
        plsc.store_scatter(msrc_b, [off + 16 * t + lanes], zeros)
        plsc.store_scatter(mdst_b, [off + 16 * t + lanes], trash)
    cbuf[...] = jnp.full((16,), off, jnp.int32)
    pltpu.sync_copy(msrc_b, msrc_hbm.at[pl.ds(wid * CAP, CAP)])
    pltpu.sync_copy(mdst_b, mdst_hbm.at[pl.ds(wid * CAP, CAP)])
    pltpu.sync_copy(cbuf, cnt_hbm.at[pl.ds(wid * 16, 16)])


@functools.cache
def _bucket():
    return pl.kernel(
        _bucket_body,
        out_type=[
            jax.ShapeDtypeStruct((NW * CAP,), jnp.int32),
            jax.ShapeDtypeStruct((NW * CAP,), jnp.int32),
            jax.ShapeDtypeStruct((NW * 16,), jnp.int32),
        ],
        mesh=_mesh(),
        compiler_params=pltpu.CompilerParams(needs_layout_passes=False),
        scratch_types=[
            pltpu.VMEM((CH,), jnp.int32),
            pltpu.VMEM((CH,), jnp.int32),
            pltpu.VMEM((CH,), jnp.int32),
            pltpu.VMEM((CH,), jnp.int32),
            pltpu.VMEM((CAP,), jnp.int32),
            pltpu.VMEM((CAP,), jnp.int32),
            pltpu.VMEM((16,), jnp.int32),
            pltpu.SemaphoreType.DMA,
            pltpu.SemaphoreType.DMA,
        ],
    )


def _segmax_body(a_hbm, b_hbm, msrc_hbm, mdst_hbm, cnt_hbm, out_hbm,
                 msrc_v, mdst_v, cnt_v, acc, acc1, rb0, rb1, bbuf, sem0, sem1):
    wid = lax.axis_index("s") * NC + lax.axis_index("c")
    lo = wid * RPT
    rows_here = jnp.minimum(RPT, N - lo)

    pltpu.sync_copy(msrc_hbm.at[pl.ds(wid * CAP, CAP)], msrc_v)
    pltpu.sync_copy(mdst_hbm.at[pl.ds(wid * CAP, CAP)], mdst_v)
    pltpu.sync_copy(cnt_hbm.at[pl.ds(wid * 16, 16)], cnt_v)
    count = jnp.minimum(jnp.max(cnt_v[...]), CAP_CLAMP)
    nq = (count + Q - 1) // Q
    npairs = (nq + 1) // 2
    qmax = jnp.maximum(nq - 1, 0)

    ninf = plsc.bitcast(jnp.full((32,), -jnp.inf, BF), jnp.int32)

    def init_body(i, _):
        acc[pl.ds(i * 16, 16)] = ninf
        acc1[pl.ds(i * 16, 16)] = ninf
        return 0

    lax.fori_loop(0, (RPT + 1) * D // 32, init_body, 0)

    def fire(q, rbuf, sem):
        idx = msrc_v.at[pl.ds(q * Q, Q)]
        pltpu.make_async_copy(a_hbm.at[idx], rbuf, sem).start()

    def wait(rbuf, sem):
        pltpu.make_async_copy(a_hbm.at[pl.ds(0, Q)], rbuf, sem).wait()

    def process(q, rbuf):
        for kk in range(Q // 16):
            vd = mdst_v[pl.ds(q * Q + kk * 16, 16)]
            bases = [vd[k] * (D // 2) for k in range(16)]
            for k in range(16):
                ac = acc if k % 2 == 0 else acc1
                for j in range(8):
                    sl = pl.ds(bases[k] + j * 16, 16)
                    av = plsc.bitcast(ac[sl], BF)
                    rv = plsc.bitcast(
                        rbuf[kk * 16 + k, pl.ds(j * 16, 16)], BF)
                    ac[sl] = plsc.bitcast(jnp.maximum(av, rv), jnp.int32)

    fire(jnp.int32(0), rb0, sem0)

    def pair_body(p, _):
        q1 = jnp.minimum(2 * p + 1, qmax)
        q2 = jnp.minimum(2 * p + 2, qmax)
        wait(rb0, sem0)
        fire(q1, rb1, sem1)
        process(2 * p, rb0)
        wait(rb1, sem1)
        fire(q2, rb0, sem0)
        process(q1, rb1)
        return 0

    lax.fori_loop(0, npairs, pair_body, 0)
    wait(rb0, sem0)

    def out_chunk(t, _):
        base_r = jnp.minimum(lo + t * 32, lo + rows_here - 32)
        pltpu.sync_copy(b_hbm.at[pl.ds(base_r, 32)], bbuf)
        arow0 = (base_r - lo) * D // 2

        lanes2 = 2 * lax.iota(jnp.int32, 16)

        def row_body(r, _):
            rows = jnp.full((16,), r, jnp.int32)
            for j in range(8):
                wsl = pl.ds(arow0 + (r * D + j * 32) // 2, 16)
                mx = jnp.maximum(plsc.bitcast(acc[wsl], BF),
                                 plsc.bitcast(acc1[wsl], BF))
                ev, od = plsc.unpack(
                    mx, format=plsc.PackFormat.INTERLEAVED,
                    preferred_element_type=jnp.float32)
                bev = bbuf[r, pl.ds(j * 32, 16)]
                bod = bbuf[r, pl.ds(j * 32 + 16, 16)]
                hev = jnp.maximum(bev + ev, 0.0)
                hod = jnp.maximum(bod + od, 0.0)
                plsc.store_scatter(bbuf, [rows, j * 32 + lanes2], hev)
                plsc.store_scatter(bbuf, [rows, j * 32 + 1 + lanes2], hod)
            return 0

        lax.fori_loop(0, 32, row_body, 0)
        pltpu.sync_copy(bbuf, out_hbm.at[pl.ds(base_r, 32)])
        return 0

    lax.fori_loop(0, RPT // 32, out_chunk, 0)


@functools.cache
def _segmax():
    return pl.kernel(
        _segmax_body,
        out_type=jax.ShapeDtypeStruct((N, D), jnp.float32),
        mesh=_mesh(),
        compiler_params=pltpu.CompilerParams(needs_layout_passes=False),
        scratch_types=[
            pltpu.VMEM((CAP,), jnp.int32),
            pltpu.VMEM((CAP,), jnp.int32),
            pltpu.VMEM((16,), jnp.int32),
            pltpu.VMEM(((RPT + 1) * D // 2,), jnp.int32),
            pltpu.VMEM(((RPT + 1) * D // 2,), jnp.int32),
            pltpu.VMEM((Q, D // 2), jnp.int32),
            pltpu.VMEM((Q, D // 2), jnp.int32),
            pltpu.VMEM((32, D), jnp.float32),
            pltpu.SemaphoreType.DMA,
            pltpu.SemaphoreType.DMA,
        ],
    )


def _to_i32(a):
    return lax.bitcast_convert_type(a.reshape(N, D // 2, 2), jnp.int32)


import numpy as _np
_PI = _np.arange(D).reshape(8, 16, 2).transpose(0, 2, 1).reshape(D)


def _unperm(h):
    return h.reshape(N, 8, 2, 16).transpose(0, 1, 3, 2).reshape(N, D)


def _prep_w(W):
    wa = W[:, :D]
    wc = W[:, D:] - wa
    wcat_t = jnp.concatenate([wa, wc], axis=0).T
    return jnp.concatenate([wcat_t[:, :D], wcat_t[:, D:][:, _PI]], axis=1)


def kernel(x, edge_index, W1, b1, W2, b2):
    src = edge_index[0]
    dst = edge_index[1]
    msrc, mdst, counts = _bucket()(src, dst)
    a1, bp1 = _matmul_ab(x, _prep_w(W1), b1[_PI].reshape(1, D))
    h1 = _segmax()(_to_i32(a1), bp1, msrc, mdst, counts)
    a2, bp2 = _matmul_ab(h1, _prep_w(W2), b2[_PI].reshape(1, D))
    h2 = _segmax()(_to_i32(a2), bp2, msrc, mdst, counts)
    return jnp.concatenate([x, h1, h2], axis=-1)


# speedup vs baseline: 1.3908x; 1.2243x over previous
import functools

import jax
import jax.numpy as jnp
from jax import lax
from jax.experimental import pallas as pl
from jax.experimental.pallas import tpu as pltpu
from jax.experimental.pallas import tpu_sc as plsc

N = 10000
E = 160000
D = 256
NC = 2
NS = 16
NW = NC * NS
RPT = 320
TRASH = RPT
CAP = 8192
Q = 16
LC = 256
CAP_CLAMP = CAP - 2 * LC - 16
CH = 8000
NCHUNK = E // CH
GPC = CH // 16
MMB = 1000
BF = jnp.bfloat16

@functools.cache
def _mesh():
    return plsc.VectorSubcoreMesh(
        core_axis_name="c", subcore_axis_name="s",
        num_cores=NC, num_subcores=NS)


def _mm_body(x_ref, w_ref, b_ref, a_ref, c_ref):
    s = jnp.dot(x_ref[...], w_ref[...], preferred_element_type=jnp.float32)
    a_ref[...] = s[:, :D].astype(BF)
    c_ref[...] = s[:, D:] + b_ref[...]


def _matmul_ab(xin, wcat_t, b2d):
    return pl.pallas_call(
        _mm_body,
        grid=(N // MMB,),
        in_specs=[
            pl.BlockSpec((MMB, D), lambda i: (i, 0)),
            pl.BlockSpec((D, 2 * D), lambda i: (0, 0)),
            pl.BlockSpec((1, D), lambda i: (0, 0)),
        ],
        out_specs=[
            pl.BlockSpec((MMB, D), lambda i: (i, 0)),
            pl.BlockSpec((MMB, D), lambda i: (i, 0)),
        ],
        out_shape=[jax.ShapeDtypeStruct((N, D), BF),
                   jax.ShapeDtypeStruct((N, D), jnp.float32)],
    )(xin, wcat_t, b2d)


def _bucket_body(src_hbm, dst_hbm, msrc_hbm, mdst_hbm, cnt_hbm,
                 sbuf0, dbuf0, sbuf1, dbuf1, msrc_b, mdst_b, cbuf,
                 csem0, csem1):
    wid = lax.axis_index("s") * NC + lax.axis_index("c")
    lo = wid * RPT
    lanes = lax.iota(jnp.int32, 16)

    def cfire(ci, sb, db, sem):
        ci = jnp.minimum(ci, NCHUNK - 1)
        pltpu.make_async_copy(src_hbm.at[pl.ds(ci * CH, CH)], sb, sem).start()
        pltpu.make_async_copy(dst_hbm.at[pl.ds(ci * CH, CH)], db, sem).start()

    def cwait(sb, db, sem):
        pltpu.make_async_copy(src_hbm.at[pl.ds(0, CH)], sb, sem).wait()
        pltpu.make_async_copy(src_hbm.at[pl.ds(0, CH)], db, sem).wait()

    def scan_chunk(sb, db, off):
        def group_body(g, off):
            vs = sb[pl.ds(g * 16, 16)]
            vd = db[pl.ds(g * 16, 16)]
            m = (vd >= lo) & (vd < lo + RPT)
            mi = m.astype(jnp.int32)
            incl = plsc.cumsum(mi)
            idx = off + incl - mi
            plsc.store_scatter(msrc_b, [idx], vs, mask=m)
            plsc.store_scatter(mdst_b, [idx], vd - lo, mask=m)
            pc = plsc.all_reduce_population_count(m)
            return jnp.minimum(off + pc[0], CAP_CLAMP)

        return lax.fori_loop(0, GPC, group_body, off)

    cfire(jnp.int32(0), sbuf0, dbuf0, csem0)

    def chunk_pair(pc, off):
        cwait(sbuf0, dbuf0, csem0)
        cfire(2 * pc + 1, sbuf1, dbuf1, csem1)
        off = scan_chunk(sbuf0, dbuf0, off)
        cwait(sbuf1, dbuf1, csem1)
        cfire(2 * pc + 2, sbuf0, dbuf0, csem0)
        off = scan_chunk(sbuf1, dbuf1, off)
        return off

    off = lax.fori_loop(0, NCHUNK // 2, chunk_pair, jnp.int32(0))
    cwait(sbuf0, dbuf0, csem0)
    zeros = jnp.zeros((16,), jnp.int32)
    trash = jnp.full((16,), TRASH, jnp.int32)
    for t in range(2 * LC // 16):
        plsc.store_scatter(msrc_b, [off + 16 * t + lanes], zeros)
        plsc.store_scatter(mdst_b, [off + 16 * t + lanes], trash)
    cbuf[...] = jnp.full((16,), off, jnp.int32)
    pltpu.sync_copy(msrc_b, msrc_hbm.at[pl.ds(wid * CAP, CAP)])
    pltpu.sync_copy(mdst_b, mdst_hbm.at[pl.ds(wid * CAP, CAP)])
    pltpu.sync_copy(cbuf, cnt_hbm.at[pl.ds(wid * 16, 16)])


@functools.cache
def _bucket():
    return pl.kernel(
        _bucket_body,
        out_type=[
            jax.ShapeDtypeStruct((NW * CAP,), jnp.int32),
            jax.ShapeDtypeStruct((NW * CAP,), jnp.int32),
            jax.ShapeDtypeStruct((NW * 16,), jnp.int32),
        ],
        mesh=_mesh(),
        compiler_params=pltpu.CompilerParams(needs_layout_passes=False),
        scratch_types=[
            pltpu.VMEM((CH,), jnp.int32),
            pltpu.VMEM((CH,), jnp.int32),
            pltpu.VMEM((CH,), jnp.int32),
            pltpu.VMEM((CH,), jnp.int32),
            pltpu.VMEM((CAP,), jnp.int32),
            pltpu.VMEM((CAP,), jnp.int32),
            pltpu.VMEM((16,), jnp.int32),
            pltpu.SemaphoreType.DMA,
            pltpu.SemaphoreType.DMA,
        ],
    )


def _segmax_body(a_hbm, b_hbm, msrc_hbm, mdst_hbm, cnt_hbm, out_hbm,
                 ls0, ld0, ls1, ld1, cnt_v, acc, ash, rb0, rb1, bbuf,
                 sem0, sem1, lsem0, lsem1):
    wid = lax.axis_index("s") * NC + lax.axis_index("c")
    sid = lax.axis_index("s")
    lo = wid * RPT
    rows_here = jnp.minimum(RPT, N - lo)

    @pl.when(sid < NS - 1)
    def _():
        pltpu.sync_copy(a_hbm.at[pl.ds(sid * 640, 640)],
                        ash.at[pl.ds(sid * 640, 640)])

    @pl.when(sid == NS - 1)
    def _():
        pltpu.sync_copy(a_hbm.at[pl.ds((NS - 1) * 640, N - (NS - 1) * 640)],
                        ash.at[pl.ds((NS - 1) * 640, N - (NS - 1) * 640)])

    pltpu.sync_copy(cnt_hbm.at[pl.ds(wid * 16, 16)], cnt_v)
    count = jnp.minimum(jnp.max(cnt_v[...]), CAP_CLAMP)
    nchunks = (count + LC - 1) // LC
    npc = (nchunks + 1) // 2
    ncmax = jnp.maximum(nchunks - 1, 0)

    ninf = plsc.bitcast(jnp.full((32,), -jnp.inf, BF), jnp.int32)

    def init_body(i, _):
        acc[pl.ds(i * 16, 16)] = ninf
        return 0

    lax.fori_loop(0, (RPT + 1) * D // 32, init_body, 0)
    plsc.subcore_barrier()

    def lfire(ci, lsb, ldb, sem):
        base = wid * CAP + jnp.minimum(ci, ncmax) * LC
        pltpu.make_async_copy(msrc_hbm.at[pl.ds(base, LC)], lsb, sem).start()
        pltpu.make_async_copy(mdst_hbm.at[pl.ds(base, LC)], ldb, sem).start()

    def lwait(lsb, ldb, sem):
        pltpu.make_async_copy(msrc_hbm.at[pl.ds(0, LC)], lsb, sem).wait()
        pltpu.make_async_copy(msrc_hbm.at[pl.ds(0, LC)], ldb, sem).wait()

    def fire(ls, gl, rbuf, sem):
        idx = ls.at[pl.ds(gl * Q, Q)]
        pltpu.make_async_copy(ash.at[idx], rbuf, sem).start()

    def wait(rbuf, sem):
        pltpu.make_async_copy(ash.at[pl.ds(0, Q)], rbuf, sem).wait()

    def process(ld, gl, rbuf):
        vd = ld[pl.ds(gl * 16, 16)]
        bases = [vd[k] * (D // 2) for k in range(16)]
        for k in range(16):
            for j in range(8):
                sl = pl.ds(bases[k] + j * 16, 16)
                av = plsc.bitcast(acc[sl], BF)
                rv = plsc.bitcast(rbuf[k, pl.ds(j * 16, 16)], BF)
                acc[sl] = plsc.bitcast(jnp.maximum(av, rv), jnp.int32)

    def chunk_process(ls, ld):
        fire(ls, 0, rb0, sem0)

        def pair_body(p2, _):
            g1 = 2 * p2 + 1
            g2 = jnp.minimum(2 * p2 + 2, LC // Q - 1)
            wait(rb0, sem0)
            fire(ls, g1, rb1, sem1)
            process(ld, 2 * p2, rb0)
            wait(rb1, sem1)
            fire(ls, g2, rb0, sem0)
            process(ld, g1, rb1)
            return 0

        lax.fori_loop(0, LC // Q // 2, pair_body, 0)
        wait(rb0, sem0)

    lfire(jnp.int32(0), ls0, ld0, lsem0)

    def opair(pp, _):
        lwait(ls0, ld0, lsem0)
        lfire(2 * pp + 1, ls1, ld1, lsem1)
        chunk_process(ls0, ld0)
        lwait(ls1, ld1, lsem1)
        lfire(2 * pp + 2, ls0, ld0, lsem0)
        chunk_process(ls1, ld1)
        return 0

    lax.fori_loop(0, npc, opair, 0)
    lwait(ls0, ld0, lsem0)

    def out_chunk(t, _):
        base_r = jnp.minimum(lo + t * 16, lo + rows_here - 16)
        pltpu.sync_copy(b_hbm.at[pl.ds(base_r, 16)], bbuf)
        arow0 = (base_r - lo) * D // 2

        lanes2 = 2 * lax.iota(jnp.int32, 16)

        def row_body(r, _):
            rows = jnp.full((16,), r, jnp.int32)
            for j in range(8):
                wsl = pl.ds(arow0 + (r * D + j * 32) // 2, 16)
                ev, od = plsc.unpack(
                    plsc.bitcast(acc[wsl], BF),
                    format=plsc.PackFormat.INTERLEAVED,
                    preferred_element_type=jnp.float32)
                bev = bbuf[r, pl.ds(j * 32, 16)]
                bod = bbuf[r, pl.ds(j * 32 + 16, 16)]
                hev = jnp.maximum(bev + ev, 0.0)
                hod = jnp.maximum(bod + od, 0.0)
                plsc.store_scatter(bbuf, [rows, j * 32 + lanes2], hev)
                plsc.store_scatter(bbuf, [rows, j * 32 + 1 + lanes2], hod)
            return 0

        lax.fori_loop(0, 16, row_body, 0)
        pltpu.sync_copy(bbuf, out_hbm.at[pl.ds(base_r, 16)])
        return 0

    lax.fori_loop(0, RPT // 16, out_chunk, 0)


@functools.cache
def _segmax():
    return pl.kernel(
        _segmax_body,
        out_type=jax.ShapeDtypeStruct((N, D), jnp.float32),
        mesh=_mesh(),
        compiler_params=pltpu.CompilerParams(needs_layout_passes=False),
        scratch_types=[
            pltpu.VMEM((LC,), jnp.int32),
            pltpu.VMEM((LC,), jnp.int32),
            pltpu.VMEM((LC,), jnp.int32),
            pltpu.VMEM((LC,), jnp.int32),
            pltpu.VMEM((16,), jnp.int32),
            pltpu.VMEM(((RPT + 1) * D // 2,), jnp.int32),
            pltpu.VMEM_SHARED((N, D // 2), jnp.int32),
            pltpu.VMEM((Q, D // 2), jnp.int32),
            pltpu.VMEM((Q, D // 2), jnp.int32),
            pltpu.VMEM((16, D), jnp.float32),
            pltpu.SemaphoreType.DMA,
            pltpu.SemaphoreType.DMA,
            pltpu.SemaphoreType.DMA,
            pltpu.SemaphoreType.DMA,
        ],
    )


def _to_i32(a):
    return lax.bitcast_convert_type(a.reshape(N, D // 2, 2), jnp.int32)


import numpy as _np
_PI = _np.arange(D).reshape(8, 16, 2).transpose(0, 2, 1).reshape(D)


def _unperm(h):
    return h.reshape(N, 8, 2, 16).transpose(0, 1, 3, 2).reshape(N, D)


def _prep_w(W):
    wa = W[:, :D]
    wc = W[:, D:] - wa
    wcat_t = jnp.concatenate([wa, wc], axis=0).T
    return jnp.concatenate([wcat_t[:, :D], wcat_t[:, D:][:, _PI]], axis=1)


def kernel(x, edge_index, W1, b1, W2, b2):
    src = edge_index[0]
    dst = edge_index[1]
    msrc, mdst, counts = _bucket()(src, dst)
    a1, bp1 = _matmul_ab(x, _prep_w(W1), b1[_PI].reshape(1, D))
    h1 = _segmax()(_to_i32(a1), bp1, msrc, mdst, counts)
    a2, bp2 = _matmul_ab(h1, _prep_w(W2), b2[_PI].reshape(1, D))
    h2 = _segmax()(_to_i32(a2), bp2, msrc, mdst, counts)
    return jnp.concatenate([x, h1, h2], axis=-1)
